# Initial kernel scaffold; baseline (speedup 1.0000x reference)
import jax
import jax.numpy as jnp
from jax.experimental import pallas as pl


def kernel(boxes, scores):
    raise NotImplementedError("write your pallas kernel here")


# speedup vs baseline: 163.0035x; 163.0035x over previous
import jax
import jax.numpy as jnp
from jax.experimental import pallas as pl

_IMG = 800.0
_MIN_SIZE = 1e-3
_TH = 0.7
_NEG = -1e9
_PRE = 2000
_PRE_PAD = 2048
_POST = 1000
_ROWB = 128


def _prep_kernel(b_ref, s_ref, bo_ref, so_ref):
    x1 = jnp.clip(b_ref[0:1, :], 0.0, _IMG)
    y1 = jnp.clip(b_ref[1:2, :], 0.0, _IMG)
    x2 = jnp.clip(b_ref[2:3, :], 0.0, _IMG)
    y2 = jnp.clip(b_ref[3:4, :], 0.0, _IMG)
    valid = ((x2 - x1) >= _MIN_SIZE) & ((y2 - y1) >= _MIN_SIZE)
    so_ref[...] = jnp.where(valid, s_ref[...], _NEG)
    bo_ref[...] = jnp.concatenate([x1, y1, x2, y2], axis=0)


def _mask_kernel(x1c_ref, y1c_ref, x2c_ref, y2c_ref,
                 x1r_ref, y1r_ref, x2r_ref, y2r_ref, m_ref):
    x1c = x1c_ref[...]
    y1c = y1c_ref[...]
    x2c = x2c_ref[...]
    y2c = y2c_ref[...]
    x1r = x1r_ref[...]
    y1r = y1r_ref[...]
    x2r = x2r_ref[...]
    y2r = y2r_ref[...]
    area_c = (x2c - x1c) * (y2c - y1c)
    area_r = (x2r - x1r) * (y2r - y1r)
    w = jnp.clip(jnp.minimum(x2c, x2r) - jnp.maximum(x1c, x1r), 0.0, None)
    h = jnp.clip(jnp.minimum(y2c, y2r) - jnp.maximum(y1c, y1r), 0.0, None)
    inter = w * h
    iou = inter / (area_c + area_r - inter + 1e-9)
    pi = pl.program_id(0)
    rowid = jax.lax.broadcasted_iota(jnp.int32, (_ROWB, _PRE_PAD), 0) + pi * _ROWB
    colid = jax.lax.broadcasted_iota(jnp.int32, (_ROWB, _PRE_PAD), 1)
    m_ref[...] = ((iou > _TH) & (rowid < colid)).astype(jnp.float32)


def _nms_kernel(m_ref, keep_ref):
    keep_ref[...] = jnp.ones((1, _PRE_PAD), dtype=jnp.float32)

    def cond(carry):
        it, changed = carry
        return changed & (it < _PRE_PAD + 1)

    def body(carry):
        it, _ = carry
        keep = keep_ref[...]
        sup = jnp.dot(keep, m_ref[...], preferred_element_type=jnp.float32)
        new = jnp.where(sup > 0.5, 0.0, 1.0)
        changed = jnp.sum(jnp.abs(new - keep)) > 0.0
        keep_ref[...] = new
        return it + 1, changed

    jax.lax.while_loop(cond, body, (jnp.int32(0), jnp.bool_(True)))


def kernel(boxes, scores):
    n = boxes.shape[0]
    n_pad = ((n + 127) // 128) * 128
    b_in = jnp.zeros((4, n_pad), dtype=jnp.float32).at[:, :n].set(boxes.T)
    s_in = jnp.full((1, n_pad), _NEG, dtype=jnp.float32).at[0, :n].set(scores)

    b_cl, s_m = pl.pallas_call(
        _prep_kernel,
        out_shape=(
            jax.ShapeDtypeStruct((4, n_pad), jnp.float32),
            jax.ShapeDtypeStruct((1, n_pad), jnp.float32),
        ),
    )(b_in, s_in)

    top_s, top_idx = jax.lax.top_k(s_m[0, :n], _PRE)
    bt = jnp.take(b_cl[:, :n], top_idx, axis=1)

    bp = jnp.zeros((4, _PRE_PAD), dtype=jnp.float32).at[:, :_PRE].set(bt)
    cols = [bp[i].reshape(_PRE_PAD, 1) for i in range(4)]
    rows = [bp[i].reshape(1, _PRE_PAD) for i in range(4)]

    m = pl.pallas_call(
        _mask_kernel,
        grid=(_PRE_PAD // _ROWB,),
        in_specs=(
            [pl.BlockSpec((_ROWB, 1), lambda i: (i, 0)) for _ in range(4)]
            + [pl.BlockSpec((1, _PRE_PAD), lambda i: (0, 0)) for _ in range(4)]
        ),
        out_specs=pl.BlockSpec((_ROWB, _PRE_PAD), lambda i: (i, 0)),
        out_shape=jax.ShapeDtypeStruct((_PRE_PAD, _PRE_PAD), jnp.float32),
    )(*cols, *rows)

    keep_f = pl.pallas_call(
        _nms_kernel,
        out_shape=jax.ShapeDtypeStruct((1, _PRE_PAD), jnp.float32),
    )(m)

    keep = keep_f[0, :_PRE] > 0.5
    final_s = jnp.where(keep, top_s, _NEG)
    out_s, fidx = jax.lax.top_k(final_s, _POST)
    out_boxes = jnp.take(bt, fidx, axis=1).T
    return out_boxes, out_s
